# baseline (speedup 1.0000x reference)
import functools

import jax
import jax.numpy as jnp
from jax import lax
from jax.experimental import pallas as pl
from jax.experimental.pallas import tpu as pltpu
from jax.experimental.pallas import tpu_sc as plsc

B = 4096
T = 50
D = 128
C = 1000
LANES = 16

NUM_WORKERS = 32
CHUNK_BAGS = 2
IDX_RAW = CHUNK_BAGS * T
IDX_PAD = 104
NVREG = D // LANES
NSPLIT = 2


def _pool(x, emb):
    bh = x.shape[0]
    bags_per_w = bh // NUM_WORKERS
    nchunk = bags_per_w // CHUNK_BAGS
    nbuf = 4

    idx = x.reshape(NUM_WORKERS, nchunk, IDX_RAW)
    npad = IDX_PAD - IDX_RAW
    padvals = jnp.arange(NUM_WORKERS * nchunk * npad,
                         dtype=jnp.int32).reshape(NUM_WORKERS, nchunk, npad)
    idx = jnp.concatenate([idx, padvals], axis=-1)

    mesh = plsc.VectorSubcoreMesh(core_axis_name="c", subcore_axis_name="s",
                                  num_cores=2, num_subcores=16)

    @functools.partial(
        pl.kernel,
        out_type=jax.ShapeDtypeStruct((bh, D), jnp.float32),
        mesh=mesh,
        scratch_types=[
            pltpu.VMEM((nchunk, IDX_PAD), jnp.int32),
            pltpu.VMEM((IDX_PAD, D), jnp.float32),
            pltpu.VMEM((IDX_PAD, D), jnp.float32),
            pltpu.VMEM((IDX_PAD, D), jnp.float32),
            pltpu.VMEM((IDX_PAD, D), jnp.float32),
            pltpu.VMEM((bags_per_w, D), jnp.float32),
            pltpu.SemaphoreType.DMA,
            pltpu.SemaphoreType.DMA,
            pltpu.SemaphoreType.DMA,
            pltpu.SemaphoreType.DMA,
        ],
    )
    def pool_kernel(idx_hbm, emb_hbm, out_hbm, idx_v, buf0, buf1, buf2,
                    buf3, out_v, sem0, sem1, sem2, sem3):
        wid = lax.axis_index("s") * 2 + lax.axis_index("c")
        pltpu.sync_copy(idx_hbm.at[wid], idx_v)
        bufs = (buf0, buf1, buf2, buf3)
        sems = (sem0, sem1, sem2, sem3)

        def start(c, b):
            pltpu.async_copy(emb_hbm.at[idx_v.at[c]], bufs[b], sems[b])

        def wait(c, b):
            pltpu.make_async_copy(emb_hbm.at[idx_v.at[c]], bufs[b],
                                  sems[b]).wait()

        def reduce_chunk(c, b):
            buf = bufs[b]
            for s in range(CHUNK_BAGS):
                def body(t, accs, _s=s):
                    return tuple(
                        accs[d] + buf[_s * T + t, pl.ds(d * LANES, LANES)]
                        for d in range(NVREG))
                accs = lax.fori_loop(
                    0, T, body,
                    tuple(jnp.zeros((LANES,), jnp.float32)
                          for _ in range(NVREG)))
                row = c * CHUNK_BAGS + s
                for d in range(NVREG):
                    out_v[row, pl.ds(d * LANES, LANES)] = (
                        accs[d] * (1.0 / T))

        for b in range(nbuf):
            start(b, b)

        def main_body(g, carry):
            for b in range(nbuf):
                c = nbuf * g + b
                wait(c, b)
                reduce_chunk(c, b)
                nxt = c + nbuf

                @pl.when(nxt < nchunk)
                def _():
                    start(nxt, b)

            return carry

        lax.fori_loop(0, nchunk // nbuf, main_body, 0)
        pltpu.sync_copy(out_v, out_hbm.at[pl.ds(wid * bags_per_w,
                                                bags_per_w)])

    return pool_kernel(idx, emb)


def _mlp(h, W1, b1, W2, b2, W3, b3, W4, b4, W5, b5):
    bh = h.shape[0]
    BM = 512

    def body(h_ref, w1, v1, w2, v2, w3, v3, w4, v4, w5, v5, o_ref):
        a = h_ref[...]
        a = jnp.maximum(
            jnp.dot(a, w1[...], preferred_element_type=jnp.float32)
            + v1[...], 0.0)
        a = jnp.maximum(
            jnp.dot(a, w2[...], preferred_element_type=jnp.float32)
            + v2[...], 0.0)
        a = jnp.maximum(
            jnp.dot(a, w3[...], preferred_element_type=jnp.float32)
            + v3[...], 0.0)
        a = jnp.maximum(
            jnp.dot(a, w4[...], preferred_element_type=jnp.float32)
            + v4[...], 0.0)
        o_ref[...] = (
            jnp.dot(a, w5[...], preferred_element_type=jnp.float32)
            + v5[...])

    full = lambda arr: pl.BlockSpec(arr.shape, lambda i: (0, 0))
    b2d = [v.reshape(1, -1) for v in (b1, b2, b3, b4, b5)]
    ws = [W1, W2, W3, W4, W5]
    in_specs = [pl.BlockSpec((BM, D), lambda i: (i, 0))]
    for w, v in zip(ws, b2d):
        in_specs.append(full(w))
        in_specs.append(full(v))

    return pl.pallas_call(
        body,
        grid=(bh // BM,),
        in_specs=in_specs,
        out_specs=pl.BlockSpec((BM, C), lambda i: (i, 0)),
        out_shape=jax.ShapeDtypeStruct((bh, C), jnp.float32),
    )(h, W1, b2d[0], W2, b2d[1], W3, b2d[2], W4, b2d[3], W5, b2d[4])


def kernel(x, emb, W1, b1, W2, b2, W3, b3, W4, b4, W5, b5):
    bh = B // NSPLIT
    pooled = [_pool(x[i * bh:(i + 1) * bh], emb) for i in range(NSPLIT)]
    outs = [_mlp(p, W1, b1, W2, b2, W3, b3, W4, b4, W5, b5)
            for p in pooled]
    return jnp.concatenate(outs, axis=0)


# speedup vs baseline: 1.0801x; 1.0801x over previous
import functools

import jax
import jax.numpy as jnp
from jax import lax
from jax.experimental import pallas as pl
from jax.experimental.pallas import tpu as pltpu
from jax.experimental.pallas import tpu_sc as plsc

B = 4096
T = 50
D = 128
C = 1000
LANES = 16

NUM_WORKERS = 32
CHUNK_BAGS = 2
IDX_RAW = CHUNK_BAGS * T
IDX_PAD = 104
NVREG = D // LANES
NSPLIT = 1


def _pool(x, emb):
    bh = x.shape[0]
    bags_per_w = bh // NUM_WORKERS
    nchunk = bags_per_w // CHUNK_BAGS
    nbuf = 4

    idx = x.reshape(NUM_WORKERS, nchunk, IDX_RAW)
    npad = IDX_PAD - IDX_RAW
    padvals = jnp.arange(NUM_WORKERS * nchunk * npad,
                         dtype=jnp.int32).reshape(NUM_WORKERS, nchunk, npad)
    idx = jnp.concatenate([idx, padvals], axis=-1)

    mesh = plsc.VectorSubcoreMesh(core_axis_name="c", subcore_axis_name="s",
                                  num_cores=2, num_subcores=16)

    @functools.partial(
        pl.kernel,
        out_type=jax.ShapeDtypeStruct((bh, D), jnp.float32),
        mesh=mesh,
        scratch_types=[
            pltpu.VMEM((nchunk, IDX_PAD), jnp.int32),
            pltpu.VMEM((IDX_PAD, D), jnp.float32),
            pltpu.VMEM((IDX_PAD, D), jnp.float32),
            pltpu.VMEM((IDX_PAD, D), jnp.float32),
            pltpu.VMEM((IDX_PAD, D), jnp.float32),
            pltpu.VMEM((bags_per_w, D), jnp.float32),
            pltpu.SemaphoreType.DMA,
            pltpu.SemaphoreType.DMA,
            pltpu.SemaphoreType.DMA,
            pltpu.SemaphoreType.DMA,
        ],
    )
    def pool_kernel(idx_hbm, emb_hbm, out_hbm, idx_v, buf0, buf1, buf2,
                    buf3, out_v, sem0, sem1, sem2, sem3):
        wid = lax.axis_index("s") * 2 + lax.axis_index("c")
        pltpu.sync_copy(idx_hbm.at[wid], idx_v)
        bufs = (buf0, buf1, buf2, buf3)
        sems = (sem0, sem1, sem2, sem3)

        def start(c, b):
            pltpu.async_copy(emb_hbm.at[idx_v.at[c]], bufs[b], sems[b])

        def wait(c, b):
            pltpu.make_async_copy(emb_hbm.at[idx_v.at[c]], bufs[b],
                                  sems[b]).wait()

        def reduce_chunk(c, b):
            buf = bufs[b]
            for s in range(CHUNK_BAGS):
                def body(t, accs, _s=s):
                    return tuple(
                        accs[d] + buf[_s * T + t, pl.ds(d * LANES, LANES)]
                        for d in range(NVREG))
                accs = lax.fori_loop(
                    0, T, body,
                    tuple(jnp.zeros((LANES,), jnp.float32)
                          for _ in range(NVREG)))
                row = c * CHUNK_BAGS + s
                for d in range(NVREG):
                    out_v[row, pl.ds(d * LANES, LANES)] = (
                        accs[d] * (1.0 / T))

        for b in range(nbuf):
            start(b, b)

        def main_body(g, carry):
            for b in range(nbuf):
                c = nbuf * g + b
                wait(c, b)
                reduce_chunk(c, b)
                nxt = c + nbuf

                @pl.when(nxt < nchunk)
                def _():
                    start(nxt, b)

            return carry

        lax.fori_loop(0, nchunk // nbuf, main_body, 0)
        pltpu.sync_copy(out_v, out_hbm.at[pl.ds(wid * bags_per_w,
                                                bags_per_w)])

    return pool_kernel(idx, emb)


def _mlp(h, W1, b1, W2, b2, W3, b3, W4, b4, W5, b5):
    bh = h.shape[0]
    BM = 512

    def body(h_ref, w1, v1, w2, v2, w3, v3, w4, v4, w5, v5, o_ref):
        a = h_ref[...]
        a = jnp.maximum(
            jnp.dot(a, w1[...], preferred_element_type=jnp.float32)
            + v1[...], 0.0)
        a = jnp.maximum(
            jnp.dot(a, w2[...], preferred_element_type=jnp.float32)
            + v2[...], 0.0)
        a = jnp.maximum(
            jnp.dot(a, w3[...], preferred_element_type=jnp.float32)
            + v3[...], 0.0)
        a = jnp.maximum(
            jnp.dot(a, w4[...], preferred_element_type=jnp.float32)
            + v4[...], 0.0)
        o_ref[...] = (
            jnp.dot(a, w5[...], preferred_element_type=jnp.float32)
            + v5[...])

    full = lambda arr: pl.BlockSpec(arr.shape, lambda i: (0, 0))
    b2d = [v.reshape(1, -1) for v in (b1, b2, b3, b4, b5)]
    ws = [W1, W2, W3, W4, W5]
    in_specs = [pl.BlockSpec((BM, D), lambda i: (i, 0))]
    for w, v in zip(ws, b2d):
        in_specs.append(full(w))
        in_specs.append(full(v))

    return pl.pallas_call(
        body,
        grid=(bh // BM,),
        in_specs=in_specs,
        out_specs=pl.BlockSpec((BM, C), lambda i: (i, 0)),
        out_shape=jax.ShapeDtypeStruct((bh, C), jnp.float32),
    )(h, W1, b2d[0], W2, b2d[1], W3, b2d[2], W4, b2d[3], W5, b2d[4])


def kernel(x, emb, W1, b1, W2, b2, W3, b3, W4, b4, W5, b5):
    bh = B // NSPLIT
    pooled = [_pool(x[i * bh:(i + 1) * bh], emb) for i in range(NSPLIT)]
    outs = [_mlp(p, W1, b1, W2, b2, W3, b3, W4, b4, W5, b5)
            for p in pooled]
    return jnp.concatenate(outs, axis=0)


# speedup vs baseline: 1.0980x; 1.0165x over previous
import functools

import jax
import jax.numpy as jnp
from jax import lax
from jax.experimental import pallas as pl
from jax.experimental.pallas import tpu as pltpu
from jax.experimental.pallas import tpu_sc as plsc

B = 4096
T = 50
D = 128
C = 1000
LANES = 16

NUM_WORKERS = 32
CHUNK_BAGS = 2
IDX_RAW = CHUNK_BAGS * T
IDX_PAD = IDX_RAW
NVREG = D // LANES
NSPLIT = 1


def _pool(x, emb):
    bh = x.shape[0]
    bags_per_w = bh // NUM_WORKERS
    nchunk = bags_per_w // CHUNK_BAGS
    nbuf = 4

    idx = x.reshape(NUM_WORKERS, nchunk, IDX_RAW)

    mesh = plsc.VectorSubcoreMesh(core_axis_name="c", subcore_axis_name="s",
                                  num_cores=2, num_subcores=16)

    @functools.partial(
        pl.kernel,
        out_type=jax.ShapeDtypeStruct((bh, D), jnp.float32),
        mesh=mesh,
        scratch_types=[
            pltpu.VMEM((nchunk, IDX_PAD), jnp.int32),
            pltpu.VMEM((IDX_PAD, D), jnp.float32),
            pltpu.VMEM((IDX_PAD, D), jnp.float32),
            pltpu.VMEM((IDX_PAD, D), jnp.float32),
            pltpu.VMEM((IDX_PAD, D), jnp.float32),
            pltpu.VMEM((bags_per_w, D), jnp.float32),
            pltpu.SemaphoreType.DMA,
            pltpu.SemaphoreType.DMA,
            pltpu.SemaphoreType.DMA,
            pltpu.SemaphoreType.DMA,
        ],
    )
    def pool_kernel(idx_hbm, emb_hbm, out_hbm, idx_v, buf0, buf1, buf2,
                    buf3, out_v, sem0, sem1, sem2, sem3):
        wid = lax.axis_index("s") * 2 + lax.axis_index("c")
        pltpu.sync_copy(idx_hbm.at[wid], idx_v)
        bufs = (buf0, buf1, buf2, buf3)
        sems = (sem0, sem1, sem2, sem3)

        def start(c, b):
            pltpu.async_copy(emb_hbm.at[idx_v.at[c]], bufs[b], sems[b])

        def wait(c, b):
            pltpu.make_async_copy(emb_hbm.at[idx_v.at[c]], bufs[b],
                                  sems[b]).wait()

        def reduce_chunk(c, b):
            buf = bufs[b]
            for s in range(CHUNK_BAGS):
                def body(t, accs, _s=s):
                    return tuple(
                        accs[d] + buf[_s * T + t, pl.ds(d * LANES, LANES)]
                        for d in range(NVREG))
                accs = lax.fori_loop(
                    0, T, body,
                    tuple(jnp.zeros((LANES,), jnp.float32)
                          for _ in range(NVREG)))
                row = c * CHUNK_BAGS + s
                for d in range(NVREG):
                    out_v[row, pl.ds(d * LANES, LANES)] = (
                        accs[d] * (1.0 / T))

        for b in range(nbuf):
            start(b, b)

        def main_body(g, carry):
            for b in range(nbuf):
                c = nbuf * g + b
                wait(c, b)
                reduce_chunk(c, b)
                nxt = c + nbuf

                @pl.when(nxt < nchunk)
                def _():
                    start(nxt, b)

            return carry

        lax.fori_loop(0, nchunk // nbuf, main_body, 0)
        pltpu.sync_copy(out_v, out_hbm.at[pl.ds(wid * bags_per_w,
                                                bags_per_w)])

    return pool_kernel(idx, emb)


def _mlp(h, W1, b1, W2, b2, W3, b3, W4, b4, W5, b5):
    bh = h.shape[0]
    BM = 512

    def body(h_ref, w1, v1, w2, v2, w3, v3, w4, v4, w5, v5, o_ref):
        a = h_ref[...]
        a = jnp.maximum(
            jnp.dot(a, w1[...], preferred_element_type=jnp.float32)
            + v1[...], 0.0)
        a = jnp.maximum(
            jnp.dot(a, w2[...], preferred_element_type=jnp.float32)
            + v2[...], 0.0)
        a = jnp.maximum(
            jnp.dot(a, w3[...], preferred_element_type=jnp.float32)
            + v3[...], 0.0)
        a = jnp.maximum(
            jnp.dot(a, w4[...], preferred_element_type=jnp.float32)
            + v4[...], 0.0)
        o_ref[...] = (
            jnp.dot(a, w5[...], preferred_element_type=jnp.float32)
            + v5[...])

    full = lambda arr: pl.BlockSpec(arr.shape, lambda i: (0, 0))
    b2d = [v.reshape(1, -1) for v in (b1, b2, b3, b4, b5)]
    ws = [W1, W2, W3, W4, W5]
    in_specs = [pl.BlockSpec((BM, D), lambda i: (i, 0))]
    for w, v in zip(ws, b2d):
        in_specs.append(full(w))
        in_specs.append(full(v))

    return pl.pallas_call(
        body,
        grid=(bh // BM,),
        in_specs=in_specs,
        out_specs=pl.BlockSpec((BM, C), lambda i: (i, 0)),
        out_shape=jax.ShapeDtypeStruct((bh, C), jnp.float32),
    )(h, W1, b2d[0], W2, b2d[1], W3, b2d[2], W4, b2d[3], W5, b2d[4])


def kernel(x, emb, W1, b1, W2, b2, W3, b3, W4, b4, W5, b5):
    bh = B // NSPLIT
    pooled = [_pool(x[i * bh:(i + 1) * bh], emb) for i in range(NSPLIT)]
    outs = [_mlp(p, W1, b1, W2, b2, W3, b3, W4, b4, W5, b5)
            for p in pooled]
    return outs[0] if NSPLIT == 1 else jnp.concatenate(outs, axis=0)
